# Initial kernel scaffold; baseline (speedup 1.0000x reference)
import jax
import jax.numpy as jnp
from jax.experimental import pallas as pl


def kernel(partial, predicted, W1, b1, W2, b2, Wd, bd, Wc, bc):
    raise NotImplementedError("write your pallas kernel here")


# speedup vs baseline: 3.2703x; 3.2703x over previous
import functools

import jax
import jax.numpy as jnp
from jax import lax
from jax.experimental import pallas as pl
from jax.experimental.pallas import tpu as pltpu

_K = 8
_QB = 128


def _fused_body(part_ref, pred_ref, w1t_ref, b1_ref, w2t_ref, b2_ref,
                wd0_ref, wd1_ref, bd_ref, wct_ref, bc_ref,
                o0_ref, o1_ref):
    part = part_ref[0]
    pred = pred_ref[0]

    qq = jnp.sum(pred * pred, axis=1, keepdims=True)
    kk = jnp.sum(part * part, axis=1)[None, :]
    qk = lax.dot_general(pred, part, (((1,), (1,)), ((), ())),
                         preferred_element_type=jnp.float32)
    d2 = qq + kk - 2.0 * qk

    def round_fn(_, carry):
        d2c, maskf = carry
        m = jnp.min(d2c, axis=1, keepdims=True)
        sel = d2c == m
        maskf = maskf + sel.astype(jnp.float32)
        d2c = jnp.where(sel, jnp.inf, d2c)
        return d2c, maskf

    _, maskf = lax.fori_loop(0, _K, round_fn, (d2, jnp.zeros_like(d2)))

    nsum = lax.dot_general(maskf, part, (((1,), (0,)), ((), ())),
                           preferred_element_type=jnp.float32)
    cnt = jnp.sum(maskf, axis=1, keepdims=True)
    nmean = nsum / cnt

    combined = jnp.concatenate([pred, nmean], axis=1)
    h = jax.nn.relu(
        lax.dot_general(combined, w1t_ref[...], (((1,), (0,)), ((), ())),
                        preferred_element_type=jnp.float32) + b1_ref[...])
    seed_feat = lax.dot_general(h, w2t_ref[...], (((1,), (0,)), ((), ())),
                                preferred_element_type=jnp.float32) + b2_ref[...]

    def head(wd_ref):
        hj = jax.nn.relu(
            lax.dot_general(seed_feat, wd_ref[...], (((1,), (0,)), ((), ())),
                            preferred_element_type=jnp.float32) + bd_ref[...])
        return lax.dot_general(hj, wct_ref[...], (((1,), (0,)), ((), ())),
                               preferred_element_type=jnp.float32) + bc_ref[...]

    o0_ref[0] = pred + head(wd0_ref)
    o1_ref[0] = pred + head(wd1_ref)


@jax.jit
def kernel(partial, predicted, W1, b1, W2, b2, Wd, bd, Wc, bc):
    B, M, _ = partial.shape
    _, N, _ = predicted.shape
    H = W1.shape[0]
    qb = min(_QB, N)

    w1t = W1.T
    w2t = W2.T
    wd0 = Wd[:, :, 0]
    wd1 = Wd[:, :, 1]
    wct = Wc.T
    b1r = b1.reshape(1, H)
    b2r = b2.reshape(1, H)
    bdr = bd.reshape(1, H)
    bcr = bc.reshape(1, 3)

    full = lambda shape: pl.BlockSpec(shape, lambda b, q: (0,) * len(shape))
    grid = (B, N // qb)
    o0, o1 = pl.pallas_call(
        _fused_body,
        grid=grid,
        in_specs=[
            pl.BlockSpec((1, M, 3), lambda b, q: (b, 0, 0)),
            pl.BlockSpec((1, qb, 3), lambda b, q: (b, q, 0)),
            full((6, H)), full((1, H)), full((H, H)), full((1, H)),
            full((H, H)), full((H, H)), full((1, H)),
            full((H, 3)), full((1, 3)),
        ],
        out_specs=[
            pl.BlockSpec((1, qb, 3), lambda b, q: (b, q, 0)),
            pl.BlockSpec((1, qb, 3), lambda b, q: (b, q, 0)),
        ],
        out_shape=[
            jax.ShapeDtypeStruct((B, N, 3), jnp.float32),
            jax.ShapeDtypeStruct((B, N, 3), jnp.float32),
        ],
        compiler_params=pltpu.CompilerParams(
            dimension_semantics=("parallel", "parallel")),
    )(partial, predicted, w1t, b1r, w2t, b2r, wd0, wd1, bdr, wct, bcr)

    out = jnp.stack([o0, o1], axis=2)
    return out.reshape(B, N * 2, 3)


# speedup vs baseline: 10.4171x; 3.1854x over previous
import jax
import jax.numpy as jnp
from jax import lax
from jax.experimental import pallas as pl
from jax.experimental.pallas import tpu as pltpu

_K = 8
_QB = 128


def _fused_body(part_ref, pred_ref, w1t_ref, b1_ref, w2t_ref, b2_ref,
                wd0_ref, wd1_ref, bd_ref, wct_ref, bc_ref,
                o0_ref, o1_ref):
    part = part_ref[0]
    pred = pred_ref[0]

    qq = jnp.sum(pred * pred, axis=1, keepdims=True)
    kk = jnp.sum(part * part, axis=1)[None, :]
    qk = lax.dot_general(pred, part, (((1,), (1,)), ((), ())),
                         preferred_element_type=jnp.float32)
    d2 = qq + kk - 2.0 * qk

    def round_fn(_, t):
        return jnp.min(jnp.where(d2 > t, d2, jnp.inf), axis=1, keepdims=True)

    t = lax.fori_loop(0, _K, round_fn,
                      jnp.full((d2.shape[0], 1), -jnp.inf, jnp.float32))
    w = (d2 <= t).astype(jnp.float32)

    nsum = lax.dot_general(w, part, (((1,), (0,)), ((), ())),
                           preferred_element_type=jnp.float32)
    cnt = jnp.sum(w, axis=1, keepdims=True)
    nmean = nsum / cnt

    combined = jnp.concatenate([pred, nmean], axis=1)
    h = jax.nn.relu(
        lax.dot_general(combined, w1t_ref[...], (((1,), (0,)), ((), ())),
                        preferred_element_type=jnp.float32) + b1_ref[...])
    seed_feat = lax.dot_general(h, w2t_ref[...], (((1,), (0,)), ((), ())),
                                preferred_element_type=jnp.float32) + b2_ref[...]

    def head(wd_ref):
        hj = jax.nn.relu(
            lax.dot_general(seed_feat, wd_ref[...], (((1,), (0,)), ((), ())),
                            preferred_element_type=jnp.float32) + bd_ref[...])
        return lax.dot_general(hj, wct_ref[...], (((1,), (0,)), ((), ())),
                               preferred_element_type=jnp.float32) + bc_ref[...]

    o0_ref[0] = pred + head(wd0_ref)
    o1_ref[0] = pred + head(wd1_ref)


@jax.jit
def kernel(partial, predicted, W1, b1, W2, b2, Wd, bd, Wc, bc):
    B, M, _ = partial.shape
    _, N, _ = predicted.shape
    H = W1.shape[0]
    qb = min(_QB, N)

    w1t = W1.T
    w2t = W2.T
    wd0 = Wd[:, :, 0]
    wd1 = Wd[:, :, 1]
    wct = Wc.T
    b1r = b1.reshape(1, H)
    b2r = b2.reshape(1, H)
    bdr = bd.reshape(1, H)
    bcr = bc.reshape(1, 3)

    full = lambda shape: pl.BlockSpec(shape, lambda b, q: (0,) * len(shape))
    grid = (B, N // qb)
    o0, o1 = pl.pallas_call(
        _fused_body,
        grid=grid,
        in_specs=[
            pl.BlockSpec((1, M, 3), lambda b, q: (b, 0, 0)),
            pl.BlockSpec((1, qb, 3), lambda b, q: (b, q, 0)),
            full((6, H)), full((1, H)), full((H, H)), full((1, H)),
            full((H, H)), full((H, H)), full((1, H)),
            full((H, 3)), full((1, 3)),
        ],
        out_specs=[
            pl.BlockSpec((1, qb, 3), lambda b, q: (b, q, 0)),
            pl.BlockSpec((1, qb, 3), lambda b, q: (b, q, 0)),
        ],
        out_shape=[
            jax.ShapeDtypeStruct((B, N, 3), jnp.float32),
            jax.ShapeDtypeStruct((B, N, 3), jnp.float32),
        ],
        compiler_params=pltpu.CompilerParams(
            dimension_semantics=("parallel", "parallel")),
    )(partial, predicted, w1t, b1r, w2t, b2r, wd0, wd1, bdr, wct, bcr)

    out = jnp.stack([o0, o1], axis=2)
    return out.reshape(B, N * 2, 3)


# speedup vs baseline: 10.9923x; 1.0552x over previous
import jax
import jax.numpy as jnp
from jax import lax
from jax.experimental import pallas as pl
from jax.experimental.pallas import tpu as pltpu

_K = 8
_QB = 128


def _fused_body(part_ref, partt_ref, pred_ref, w1t_ref, b1_ref, w2t_ref,
                b2_ref, wd0_ref, wd1_ref, bd_ref, wct_ref, bc_ref,
                o0_ref, o1_ref):
    part = part_ref[0]
    partt = partt_ref[0]
    pred = pred_ref[0]

    qq = jnp.sum(pred * pred, axis=1, keepdims=True)
    kk = jnp.sum(partt * partt, axis=0, keepdims=True)
    qk = lax.dot_general(pred, partt, (((1,), (0,)), ((), ())),
                         preferred_element_type=jnp.float32)
    d2 = qq + kk - 2.0 * qk

    def round_fn(_, t):
        return jnp.min(jnp.where(d2 > t, d2, jnp.inf), axis=1, keepdims=True)

    t = lax.fori_loop(0, _K, round_fn,
                      jnp.full((d2.shape[0], 1), -jnp.inf, jnp.float32))
    w = (d2 <= t).astype(jnp.float32)

    nsum = lax.dot_general(w, part, (((1,), (0,)), ((), ())),
                           preferred_element_type=jnp.float32)
    cnt = jnp.sum(w, axis=1, keepdims=True)
    nmean = nsum / cnt

    combined = jnp.concatenate([pred, nmean], axis=1)
    h = jax.nn.relu(
        lax.dot_general(combined, w1t_ref[...], (((1,), (0,)), ((), ())),
                        preferred_element_type=jnp.float32) + b1_ref[...])
    seed_feat = lax.dot_general(h, w2t_ref[...], (((1,), (0,)), ((), ())),
                                preferred_element_type=jnp.float32) + b2_ref[...]

    def head(wd_ref):
        hj = jax.nn.relu(
            lax.dot_general(seed_feat, wd_ref[...], (((1,), (0,)), ((), ())),
                            preferred_element_type=jnp.float32) + bd_ref[...])
        return lax.dot_general(hj, wct_ref[...], (((1,), (0,)), ((), ())),
                               preferred_element_type=jnp.float32) + bc_ref[...]

    o0_ref[0] = pred + head(wd0_ref)
    o1_ref[0] = pred + head(wd1_ref)


@jax.jit
def kernel(partial, predicted, W1, b1, W2, b2, Wd, bd, Wc, bc):
    B, M, _ = partial.shape
    _, N, _ = predicted.shape
    H = W1.shape[0]
    qb = min(_QB, N)

    w1t = W1.T
    w2t = W2.T
    wd0 = Wd[:, :, 0]
    wd1 = Wd[:, :, 1]
    wct = Wc.T
    b1r = b1.reshape(1, H)
    b2r = b2.reshape(1, H)
    bdr = bd.reshape(1, H)
    bcr = bc.reshape(1, 3)

    full = lambda shape: pl.BlockSpec(shape, lambda b, q: (0,) * len(shape))
    grid = (B, N // qb)
    o0, o1 = pl.pallas_call(
        _fused_body,
        grid=grid,
        in_specs=[
            pl.BlockSpec((1, M, 3), lambda b, q: (b, 0, 0)),
            pl.BlockSpec((1, 3, M), lambda b, q: (b, 0, 0)),
            pl.BlockSpec((1, qb, 3), lambda b, q: (b, q, 0)),
            full((6, H)), full((1, H)), full((H, H)), full((1, H)),
            full((H, H)), full((H, H)), full((1, H)),
            full((H, 3)), full((1, 3)),
        ],
        out_specs=[
            pl.BlockSpec((1, qb, 3), lambda b, q: (b, q, 0)),
            pl.BlockSpec((1, qb, 3), lambda b, q: (b, q, 0)),
        ],
        out_shape=[
            jax.ShapeDtypeStruct((B, N, 3), jnp.float32),
            jax.ShapeDtypeStruct((B, N, 3), jnp.float32),
        ],
        compiler_params=pltpu.CompilerParams(
            dimension_semantics=("parallel", "parallel")),
    )(partial, partial.transpose(0, 2, 1), predicted,
      w1t, b1r, w2t, b2r, wd0, wd1, bdr, wct, bcr)

    out = jnp.stack([o0, o1], axis=2)
    return out.reshape(B, N * 2, 3)
